# baseline (speedup 1.0000x reference)
import functools

import jax
import jax.numpy as jnp
import numpy as np
from jax.experimental import pallas as pl
from jax.experimental.pallas import tpu as pltpu
from jax.experimental.pallas import tpu_sc as plsc

_EPS = 1e-5
_BLK_S = 2048


def _ln_body(x_ref, pos_ref, gamma_ref, beta_ref, out_ref):
    h = x_ref[0] + pos_ref[...]
    mean = jnp.mean(h, axis=-1, keepdims=True)
    c = h - mean
    var = jnp.mean(c * c, axis=-1, keepdims=True)
    inv = jax.lax.rsqrt(var + _EPS)
    out_ref[0] = c * inv * gamma_ref[...] + beta_ref[...]


@jax.jit
def _ln_tc(x, pos_table, gamma, beta):
    b, s, d = x.shape
    grid = (s // _BLK_S, b)
    return pl.pallas_call(
        _ln_body,
        grid=grid,
        in_specs=[
            pl.BlockSpec((1, _BLK_S, d), lambda i, j: (j, i, 0)),
            pl.BlockSpec((_BLK_S, d), lambda i, j: (i, 0)),
            pl.BlockSpec((1, d), lambda i, j: (0, 0)),
            pl.BlockSpec((1, d), lambda i, j: (0, 0)),
        ],
        out_specs=pl.BlockSpec((1, _BLK_S, d), lambda i, j: (j, i, 0)),
        out_shape=jax.ShapeDtypeStruct((b, s, d), x.dtype),
    )(x, pos_table, gamma.reshape(1, d), beta.reshape(1, d))


_NC = 2
_NS = 16
_L = 16
_RCHUNK = 16


def _xlane_sum(v):
    dnums = jax.lax.GatherDimensionNumbers(
        offset_dims=(), collapsed_slice_dims=(0,), start_index_map=(0,))
    lane = jax.lax.iota(jnp.int32, _L)
    for k in (1, 2, 4, 8):
        idx = jnp.reshape(lane ^ k, (_L, 1))
        v = v + jax.lax.gather(
            v, idx, dnums, slice_sizes=(1,),
            mode=jax.lax.GatherScatterMode.PROMISE_IN_BOUNDS)
    return v


def _newton_rsqrt(v):
    i = jax.lax.bitcast_convert_type(v, jnp.int32)
    i = jnp.int32(0x5F3759DF) - jax.lax.shift_right_logical(i, 1)
    y = jax.lax.bitcast_convert_type(i, jnp.float32)
    for _ in range(3):
        y = y * (1.5 - 0.5 * v * y * y)
    return y


def _sc_body(x_hbm, pos_hbm, gamma_hbm, beta_hbm, out_hbm,
             xbuf, posbuf, gbuf, bbuf):
    d = gbuf.shape[0]
    nj = d // _L
    n_rows, _ = x_hbm.shape
    n_seq, _ = pos_hbm.shape
    batch = n_rows // n_seq
    seq_per_tile = n_seq // (_NC * _NS)
    n_chunks = seq_per_tile // _RCHUNK

    cid = jax.lax.axis_index("c")
    sid = jax.lax.axis_index("s")
    wid = sid * _NC + cid
    seq_base = wid * seq_per_tile

    pltpu.sync_copy(gamma_hbm, gbuf)
    pltpu.sync_copy(beta_hbm, bbuf)

    def chunk_body(c, carry):
        p0 = seq_base + c * _RCHUNK
        pltpu.sync_copy(pos_hbm.at[pl.ds(p0, _RCHUNK)], posbuf)
        for b in range(batch):
            row0 = b * n_seq + p0
            pltpu.sync_copy(x_hbm.at[pl.ds(row0, _RCHUNK)], xbuf)

            def row_body(r, rcarry):
                acc = jnp.zeros((_L,), jnp.float32)
                acc2 = jnp.zeros((_L,), jnp.float32)
                for j in range(nj):
                    sl = pl.ds(j * _L, _L)
                    v = xbuf[r, sl] + posbuf[r, sl]
                    xbuf[r, sl] = v
                    acc = acc + v
                    acc2 = acc2 + v * v
                mean_v = _xlane_sum(acc) * (1.0 / d)
                ex2_v = _xlane_sum(acc2) * (1.0 / d)
                var_v = ex2_v - mean_v * mean_v + _EPS
                rstd = _newton_rsqrt(var_v)
                for j in range(nj):
                    sl = pl.ds(j * _L, _L)
                    v = (xbuf[r, sl] - mean_v) * rstd * gbuf[sl] + bbuf[sl]
                    xbuf[r, sl] = v
                return rcarry

            jax.lax.fori_loop(0, _RCHUNK, row_body, 0)
            pltpu.sync_copy(xbuf, out_hbm.at[pl.ds(row0, _RCHUNK)])
        return carry

    jax.lax.fori_loop(0, n_chunks, chunk_body, 0)


@jax.jit
def _ln_sc(x, pos_table, gamma, beta):
    b, s, d = x.shape
    x2 = x.reshape(b * s, d)
    mesh = plsc.VectorSubcoreMesh(core_axis_name="c", subcore_axis_name="s")
    out = pl.kernel(
        _sc_body,
        out_type=jax.ShapeDtypeStruct((b * s, d), x.dtype),
        mesh=mesh,
        scratch_types=[
            pltpu.VMEM((_RCHUNK, d), jnp.float32),
            pltpu.VMEM((_RCHUNK, d), jnp.float32),
            pltpu.VMEM((d,), jnp.float32),
            pltpu.VMEM((d,), jnp.float32),
        ],
    )(x2, pos_table, gamma, beta)
    return out.reshape(b, s, d)


def kernel(x, pos_table, gamma, beta, batch_size):
    return _ln_sc(x, pos_table, gamma, beta)
